# baseline (speedup 1.0000x reference)
import functools

import jax
import jax.numpy as jnp
from jax import lax
from jax.experimental import pallas as pl
from jax.experimental.pallas import tpu as pltpu
from jax.experimental.pallas import tpu_sc as plsc

N_NODES = 10000
N_EDGES = 160000
D_EDGE = 16
LANES = 16
N_WORKERS = 16
EPW = N_EDGES // N_WORKERS
GROUPS = EPW // LANES
NPAD = 10240
SEG_PW = NPAD // N_WORKERS
NEG_INF = float("-inf")


def _prep_body(w_ref, x_ref, idx_ref, a_ref, src_ref):
    y = lax.dot_general(w_ref[...], x_ref[...], (((0,), (0,)), ((), ())),
                        preferred_element_type=jnp.float32)
    a_ref[...] = y[0]
    src_ref[...] = idx_ref[0]


def _edge_prep(edge_features, edge_index, W_attn):
    eft = edge_features.T
    return pl.pallas_call(
        _prep_body,
        out_shape=[
            jax.ShapeDtypeStruct((N_EDGES,), jnp.float32),
            jax.ShapeDtypeStruct((N_EDGES,), jnp.int32),
        ],
    )(W_attn, eft, edge_index)


def _sc_body(a_hbm, src_hbm, out_hbm,
             a_v, src_v, ex_v, pmax_v, glob_v, red_v, gseg_v,
             kbuf, vbuf, accb, gmx_v,
             pmax_sh, gmax_sh, den_sh, red16_sh):
    wid = lax.axis_index("s")
    base_e = wid * EPW
    pltpu.sync_copy(a_hbm.at[pl.ds(base_e, EPW)], a_v)
    pltpu.sync_copy(src_hbm.at[pl.ds(base_e, EPW)], src_v)

    neg = jnp.full((LANES,), NEG_INF, jnp.float32)
    zeros = jnp.zeros((LANES,), jnp.float32)
    iot = lax.iota(jnp.int32, LANES)

    def lmax_body(i, m):
        return jnp.maximum(m, a_v[pl.ds(i * LANES, LANES)])
    m16 = lax.fori_loop(0, GROUPS, lmax_body, neg)
    vbuf[...] = m16
    pltpu.sync_copy(vbuf, red16_sh.at[wid])

    def zero_body(j, _):
        gseg_v[pl.ds(j * LANES, LANES)] = zeros
        return _
    lax.fori_loop(0, SEG_PW // LANES, zero_body, None)
    pltpu.sync_copy(gseg_v, den_sh.at[pl.ds(wid * SEG_PW, SEG_PW)])
    plsc.subcore_barrier()

    pltpu.sync_copy(red16_sh, gmx_v)
    cm = neg
    for r in range(N_WORKERS):
        cm = jnp.maximum(cm, gmx_v[r, :])
    c_glob = lax.reduce_max(cm, axes=(0,))

    def qb_body(i, _):
        b = i * LANES
        ex_v[pl.ds(b, LANES)] = jnp.exp(a_v[pl.ds(b, LANES)] - c_glob)
        return _
    lax.fori_loop(0, GROUPS, qb_body, None)
    pltpu.sync_copy(ex_v, den_sh.at[src_v], add=True)
    plsc.subcore_barrier()
    pltpu.sync_copy(den_sh, glob_v)

    def qc_body(i, carry):
        acc, bad = carry
        b = i * LANES
        s16 = src_v[pl.ds(b, LANES)]
        e16 = ex_v[pl.ds(b, LANES)]
        d16 = plsc.load_gather(glob_v, [s16])
        ok = d16 > 0.0
        acc = acc + jnp.where(ok, e16 / d16, 0.0)
        bad = bad | ~ok
        return acc, bad
    qacc, qbad = lax.fori_loop(0, GROUPS, qc_body,
                               (jnp.zeros((LANES,), jnp.float32),
                                jnp.zeros((LANES,), jnp.bool_)))
    accb[...] = qacc
    vbuf[...] = jnp.where(qbad, 1.0, 0.0)
    pltpu.sync_copy(vbuf, red16_sh.at[wid])
    plsc.subcore_barrier()
    pltpu.sync_copy(red16_sh, gmx_v)
    fb = zeros
    for r in range(N_WORKERS):
        fb = jnp.maximum(fb, gmx_v[r, :])
    need_exact = lax.reduce_max(fb, axes=(0,)) > 0.0

    @pl.when(need_exact)
    def _fallback():
        def init_body(i, _):
            pmax_v[pl.ds(i * LANES, LANES)] = neg
            return _
        lax.fori_loop(0, NPAD // LANES, init_body, None)

        def phase_a(i, _):
            b = i * LANES
            s16 = src_v[pl.ds(b, LANES)]
            a16 = a_v[pl.ds(b, LANES)]
            sk, sv = plsc.sort_key_val(s16, a16)
            kbuf[...] = sk
            m = sv
            for k in (1, 2, 4, 8):
                j = jnp.maximum(iot - k, 0)
                vbuf[...] = m
                pm = plsc.load_gather(vbuf, [j])
                ps = plsc.load_gather(kbuf, [j])
                take = (ps == sk) & (iot >= k)
                m = jnp.where(take, jnp.maximum(m, pm), m)
            ns = plsc.load_gather(kbuf, [jnp.minimum(iot + 1, LANES - 1)])
            last = (ns != sk) | (iot == LANES - 1)
            old = plsc.load_gather(pmax_v, [sk])
            plsc.store_scatter(pmax_v, [sk], jnp.maximum(old, m), mask=last)
            return _
        lax.fori_loop(0, GROUPS, phase_a, None)

        pltpu.sync_copy(pmax_v, pmax_sh.at[wid])
        plsc.subcore_barrier()
        seg_lo = wid * SEG_PW
        pltpu.sync_copy(pmax_sh.at[:, pl.ds(seg_lo, SEG_PW)], red_v)

        def red_body(j, _):
            cc = j * LANES
            m = red_v[0, pl.ds(cc, LANES)]
            for r in range(1, N_WORKERS):
                m = jnp.maximum(m, red_v[r, pl.ds(cc, LANES)])
            gseg_v[pl.ds(cc, LANES)] = m
            return _
        lax.fori_loop(0, SEG_PW // LANES, red_body, None)
        pltpu.sync_copy(gseg_v, gmax_sh.at[pl.ds(seg_lo, SEG_PW)])

        def zero2_body(j, _):
            gseg_v[pl.ds(j * LANES, LANES)] = zeros
            return _
        lax.fori_loop(0, SEG_PW // LANES, zero2_body, None)
        pltpu.sync_copy(gseg_v, den_sh.at[pl.ds(seg_lo, SEG_PW)])
        plsc.subcore_barrier()
        pltpu.sync_copy(gmax_sh, glob_v)

        def phase_b(i, _):
            b = i * LANES
            s16 = src_v[pl.ds(b, LANES)]
            a16 = a_v[pl.ds(b, LANES)]
            mx = plsc.load_gather(glob_v, [s16])
            ex_v[pl.ds(b, LANES)] = jnp.exp(a16 - mx)
            return _
        lax.fori_loop(0, GROUPS, phase_b, None)
        pltpu.sync_copy(ex_v, den_sh.at[src_v], add=True)
        plsc.subcore_barrier()
        pltpu.sync_copy(den_sh, glob_v)

        def phase_c(i, acc):
            b = i * LANES
            s16 = src_v[pl.ds(b, LANES)]
            e16 = ex_v[pl.ds(b, LANES)]
            d16 = plsc.load_gather(glob_v, [s16])
            return acc + e16 / d16
        acc = lax.fori_loop(0, GROUPS, phase_c,
                            jnp.zeros((LANES,), jnp.float32))
        accb[...] = acc

    pltpu.sync_copy(accb, out_hbm.at[wid])


_sc_softmax_partials = functools.partial(
    pl.kernel,
    mesh=plsc.VectorSubcoreMesh(core_axis_name="c", subcore_axis_name="s",
                                num_cores=1),
    compiler_params=pltpu.CompilerParams(needs_layout_passes=False),
    out_type=jax.ShapeDtypeStruct((N_WORKERS, LANES), jnp.float32),
    scratch_types=[
        pltpu.VMEM((EPW,), jnp.float32),
        pltpu.VMEM((EPW,), jnp.int32),
        pltpu.VMEM((EPW,), jnp.float32),
        pltpu.VMEM((NPAD,), jnp.float32),
        pltpu.VMEM((NPAD,), jnp.float32),
        pltpu.VMEM((N_WORKERS, SEG_PW), jnp.float32),
        pltpu.VMEM((SEG_PW,), jnp.float32),
        pltpu.VMEM((LANES,), jnp.int32),
        pltpu.VMEM((LANES,), jnp.float32),
        pltpu.VMEM((LANES,), jnp.float32),
        pltpu.VMEM((N_WORKERS, LANES), jnp.float32),
        pltpu.VMEM_SHARED((N_WORKERS, NPAD), jnp.float32),
        pltpu.VMEM_SHARED((NPAD,), jnp.float32),
        pltpu.VMEM_SHARED((NPAD,), jnp.float32),
        pltpu.VMEM_SHARED((N_WORKERS, LANES), jnp.float32),
    ],
)(_sc_body)


def _h_body(p_ref, x_ref, o_ref):
    o_ref[...] = x_ref[...] + 0.0 * jnp.sum(p_ref[...])


def kernel(node_features, edge_features, edge_index, W_attn):
    a, src = _edge_prep(edge_features, edge_index.astype(jnp.int32), W_attn)
    partials = _sc_softmax_partials(a, src)

    rows, cols = node_features.shape
    blk = 2000
    h = pl.pallas_call(
        _h_body,
        grid=(rows // blk,),
        in_specs=[
            pl.BlockSpec((N_WORKERS, LANES), lambda i: (0, 0)),
            pl.BlockSpec((blk, cols), lambda i: (i, 0)),
        ],
        out_specs=pl.BlockSpec((blk, cols), lambda i: (i, 0)),
        out_shape=jax.ShapeDtypeStruct(node_features.shape,
                                       node_features.dtype),
    )(partials, node_features)
    return h


# speedup vs baseline: 1.2654x; 1.2654x over previous
import functools

import jax
import jax.numpy as jnp
from jax import lax
from jax.experimental import pallas as pl
from jax.experimental.pallas import tpu as pltpu
from jax.experimental.pallas import tpu_sc as plsc

N_NODES = 10000
N_EDGES = 160000
D_EDGE = 16
LANES = 16
N_WORKERS = 16
EPW = N_EDGES // N_WORKERS
GROUPS = EPW // LANES
UNROLL = 5
NPAD = 10240
SEG_PW = NPAD // N_WORKERS
NEG_INF = float("-inf")


def _prep_body(w_ref, x_ref, idx_ref, a_ref, src_ref):
    y = lax.dot_general(w_ref[...], x_ref[...], (((0,), (0,)), ((), ())),
                        preferred_element_type=jnp.float32)
    a_ref[...] = y[0]
    src_ref[...] = idx_ref[0]


def _edge_prep(edge_features, edge_index, W_attn):
    eft = edge_features.T
    return pl.pallas_call(
        _prep_body,
        out_shape=[
            jax.ShapeDtypeStruct((N_EDGES,), jnp.float32),
            jax.ShapeDtypeStruct((N_EDGES,), jnp.int32),
        ],
    )(W_attn, eft, edge_index)


def _sc_body(a_hbm, src_hbm, out_hbm,
             a_v, src_v, ex_v, pmax_v, glob_v, red_v, gseg_v,
             kb0, kb1, kb2, kb3, kb4, vb0, vb1, vb2, vb3, vb4, accb,
             pmax_sh, gmax_sh, den_sh):
    kbufs = (kb0, kb1, kb2, kb3, kb4)
    vbufs = (vb0, vb1, vb2, vb3, vb4)
    wid = lax.axis_index("s")
    base_e = wid * EPW
    with jax.named_scope("sc_dma_in"):
        pltpu.sync_copy(a_hbm.at[pl.ds(base_e, EPW)], a_v)
        pltpu.sync_copy(src_hbm.at[pl.ds(base_e, EPW)], src_v)

    neg = jnp.full((LANES,), NEG_INF, jnp.float32)
    iot = lax.iota(jnp.int32, LANES)

    with jax.named_scope("sc_init"):
        def init_body(i, _):
            for u in range(8):
                pmax_v[pl.ds((i * 8 + u) * LANES, LANES)] = neg
            return _
        lax.fori_loop(0, NPAD // LANES // 8, init_body, None)

    with jax.named_scope("sc_phase_a"):
        def phase_a(i, _):
            for u in range(UNROLL):
                kbuf, vbuf = kbufs[u], vbufs[u]
                b = (i * UNROLL + u) * LANES
                s16 = src_v[pl.ds(b, LANES)]
                a16 = a_v[pl.ds(b, LANES)]
                sk, sv = plsc.sort_key_val(s16, a16)
                kbuf[...] = sk
                m = sv
                for k in (1, 2, 4, 8):
                    j = jnp.maximum(iot - k, 0)
                    vbuf[...] = m
                    pm = plsc.load_gather(vbuf, [j])
                    ps = plsc.load_gather(kbuf, [j])
                    take = (ps == sk) & (iot >= k)
                    m = jnp.where(take, jnp.maximum(m, pm), m)
                ns = plsc.load_gather(kbuf, [jnp.minimum(iot + 1, LANES - 1)])
                last = (ns != sk) | (iot == LANES - 1)
                old = plsc.load_gather(pmax_v, [sk])
                plsc.store_scatter(pmax_v, [sk], jnp.maximum(old, m),
                                   mask=last)
            return _
        lax.fori_loop(0, GROUPS // UNROLL, phase_a, None)

    with jax.named_scope("sc_combine_max"):
        pltpu.sync_copy(pmax_v, pmax_sh.at[wid])
        plsc.subcore_barrier()
        seg_lo = wid * SEG_PW
        pltpu.sync_copy(pmax_sh.at[:, pl.ds(seg_lo, SEG_PW)], red_v)

        def red_body(j, _):
            cc = j * LANES
            m = red_v[0, pl.ds(cc, LANES)]
            for r in range(1, N_WORKERS):
                m = jnp.maximum(m, red_v[r, pl.ds(cc, LANES)])
            gseg_v[pl.ds(cc, LANES)] = m
            return _
        lax.fori_loop(0, SEG_PW // LANES, red_body, None)
        pltpu.sync_copy(gseg_v, gmax_sh.at[pl.ds(seg_lo, SEG_PW)])

        zeros = jnp.zeros((LANES,), jnp.float32)

        def zero_body(j, _):
            for u in range(8):
                gseg_v[pl.ds((j * 8 + u) * LANES, LANES)] = zeros
            return _
        lax.fori_loop(0, SEG_PW // LANES // 8, zero_body, None)
        pltpu.sync_copy(gseg_v, den_sh.at[pl.ds(seg_lo, SEG_PW)])
        plsc.subcore_barrier()
        pltpu.sync_copy(gmax_sh, glob_v)

    with jax.named_scope("sc_phase_b"):
        def phase_b(i, _):
            for u in range(UNROLL):
                b = (i * UNROLL + u) * LANES
                s16 = src_v[pl.ds(b, LANES)]
                a16 = a_v[pl.ds(b, LANES)]
                mx = plsc.load_gather(glob_v, [s16])
                ex_v[pl.ds(b, LANES)] = jnp.exp(a16 - mx)
            return _
        lax.fori_loop(0, GROUPS // UNROLL, phase_b, None)
    with jax.named_scope("sc_scatter_add"):
        pltpu.sync_copy(ex_v, den_sh.at[src_v], add=True)
        plsc.subcore_barrier()
        pltpu.sync_copy(den_sh, glob_v)

    with jax.named_scope("sc_phase_c"):
        def phase_c(i, acc):
            for u in range(UNROLL):
                b = (i * UNROLL + u) * LANES
                s16 = src_v[pl.ds(b, LANES)]
                e16 = ex_v[pl.ds(b, LANES)]
                d16 = plsc.load_gather(glob_v, [s16])
                acc = acc + e16 / d16
            return acc
        acc = lax.fori_loop(0, GROUPS // UNROLL, phase_c,
                            jnp.zeros((LANES,), jnp.float32))
        accb[...] = acc
        pltpu.sync_copy(accb, out_hbm.at[wid])


_sc_softmax_partials = functools.partial(
    pl.kernel,
    mesh=plsc.VectorSubcoreMesh(core_axis_name="c", subcore_axis_name="s",
                                num_cores=1),
    compiler_params=pltpu.CompilerParams(needs_layout_passes=False),
    out_type=jax.ShapeDtypeStruct((N_WORKERS, LANES), jnp.float32),
    scratch_types=[
        pltpu.VMEM((EPW,), jnp.float32),
        pltpu.VMEM((EPW,), jnp.int32),
        pltpu.VMEM((EPW,), jnp.float32),
        pltpu.VMEM((NPAD,), jnp.float32),
        pltpu.VMEM((NPAD,), jnp.float32),
        pltpu.VMEM((N_WORKERS, SEG_PW), jnp.float32),
        pltpu.VMEM((SEG_PW,), jnp.float32),
        pltpu.VMEM((LANES,), jnp.int32),
        pltpu.VMEM((LANES,), jnp.int32),
        pltpu.VMEM((LANES,), jnp.int32),
        pltpu.VMEM((LANES,), jnp.int32),
        pltpu.VMEM((LANES,), jnp.int32),
        pltpu.VMEM((LANES,), jnp.float32),
        pltpu.VMEM((LANES,), jnp.float32),
        pltpu.VMEM((LANES,), jnp.float32),
        pltpu.VMEM((LANES,), jnp.float32),
        pltpu.VMEM((LANES,), jnp.float32),
        pltpu.VMEM((LANES,), jnp.float32),
        pltpu.VMEM_SHARED((N_WORKERS, NPAD), jnp.float32),
        pltpu.VMEM_SHARED((NPAD,), jnp.float32),
        pltpu.VMEM_SHARED((NPAD,), jnp.float32),
    ],
)(_sc_body)


def _h_body(p_ref, x_ref, o_ref):
    o_ref[...] = x_ref[...] + 0.0 * jnp.sum(p_ref[...])


def kernel(node_features, edge_features, edge_index, W_attn):
    a, src = _edge_prep(edge_features, edge_index.astype(jnp.int32), W_attn)
    partials = _sc_softmax_partials(a, src)

    rows, cols = node_features.shape
    blk = 2000
    h = pl.pallas_call(
        _h_body,
        grid=(rows // blk,),
        in_specs=[
            pl.BlockSpec((N_WORKERS, LANES), lambda i: (0, 0)),
            pl.BlockSpec((blk, cols), lambda i: (i, 0)),
        ],
        out_specs=pl.BlockSpec((blk, cols), lambda i: (i, 0)),
        out_shape=jax.ShapeDtypeStruct(node_features.shape,
                                       node_features.dtype),
    )(partials, node_features)
    return h


# speedup vs baseline: 2.1655x; 1.7114x over previous
import functools

import jax
import jax.numpy as jnp
from jax import lax
from jax.experimental import pallas as pl
from jax.experimental.pallas import tpu as pltpu
from jax.experimental.pallas import tpu_sc as plsc

N_NODES = 10000
N_EDGES = 160000
D_EDGE = 16
LANES = 16
N_WORKERS = 16
EPW = N_EDGES // N_WORKERS
GROUPS = EPW // LANES
UNROLL = 5
NPAD = 10240
SEG_PW = NPAD // N_WORKERS
NEG_INF = float("-inf")


def _prep_body(w_ref, x_ref, idx_ref, a_ref, src_ref):
    y = lax.dot_general(w_ref[...], x_ref[...], (((0,), (0,)), ((), ())),
                        preferred_element_type=jnp.float32)
    a_ref[...] = y[0]
    src_ref[...] = idx_ref[0]


def _edge_prep(edge_features, edge_index, W_attn):
    eft = edge_features.T
    return pl.pallas_call(
        _prep_body,
        out_shape=[
            jax.ShapeDtypeStruct((N_EDGES,), jnp.float32),
            jax.ShapeDtypeStruct((N_EDGES,), jnp.int32),
        ],
    )(W_attn, eft, edge_index)


def _sc_body(a_hbm, src_hbm, out_hbm,
             a_v, src_v, ex_v, pmax_v, glob_v, red_v, gseg_v,
             kb0, kb1, kb2, kb3, kb4, vb0, vb1, vb2, vb3, vb4, accb,
             pmax_sh, gmax_sh, den_sh):
    kbufs = (kb0, kb1, kb2, kb3, kb4)
    vbufs = (vb0, vb1, vb2, vb3, vb4)
    wid = lax.axis_index("s")
    base_e = wid * EPW
    with jax.named_scope("sc_dma_in"):
        pltpu.sync_copy(a_hbm.at[pl.ds(base_e, EPW)], a_v)
        pltpu.sync_copy(src_hbm.at[pl.ds(base_e, EPW)], src_v)

    neg = jnp.full((LANES,), NEG_INF, jnp.float32)
    iot = lax.iota(jnp.int32, LANES)

    with jax.named_scope("sc_init"):
        def init_body(i, _):
            for u in range(8):
                pmax_v[pl.ds((i * 8 + u) * LANES, LANES)] = neg
            return _
        lax.fori_loop(0, NPAD // LANES // 8, init_body, None)

    with jax.named_scope("sc_phase_a"):
        def phase_a(i, _):
            for u in range(UNROLL):
                kbuf, vbuf = kbufs[u], vbufs[u]
                b = (i * UNROLL + u) * LANES
                s16 = src_v[pl.ds(b, LANES)]
                a16 = a_v[pl.ds(b, LANES)]
                sk, sv = plsc.sort_key_val(s16, a16)
                kbuf[...] = sk
                m = sv
                for k in (1, 2, 4, 8):
                    j = jnp.maximum(iot - k, 0)
                    vbuf[...] = m
                    pm = plsc.load_gather(vbuf, [j])
                    ps = plsc.load_gather(kbuf, [j])
                    take = (ps == sk) & (iot >= k)
                    m = jnp.where(take, jnp.maximum(m, pm), m)
                ns = plsc.load_gather(kbuf, [jnp.minimum(iot + 1, LANES - 1)])
                last = (ns != sk) | (iot == LANES - 1)
                old = plsc.load_gather(pmax_v, [sk])
                plsc.store_scatter(pmax_v, [sk], jnp.maximum(old, m),
                                   mask=last)
            return _
        lax.fori_loop(0, GROUPS // UNROLL, phase_a, None)

    with jax.named_scope("sc_combine_max"):
        pltpu.sync_copy(pmax_v, pmax_sh.at[wid])
        plsc.subcore_barrier()
        seg_lo = wid * SEG_PW
        pltpu.sync_copy(pmax_sh.at[:, pl.ds(seg_lo, SEG_PW)], red_v)

        def red_body(j, _):
            cc = j * LANES
            m = red_v[0, pl.ds(cc, LANES)]
            for r in range(1, N_WORKERS):
                m = jnp.maximum(m, red_v[r, pl.ds(cc, LANES)])
            gseg_v[pl.ds(cc, LANES)] = m
            return _
        lax.fori_loop(0, SEG_PW // LANES, red_body, None)
        pltpu.sync_copy(gseg_v, gmax_sh.at[pl.ds(seg_lo, SEG_PW)])

        zeros = jnp.zeros((LANES,), jnp.float32)

        def zero_body(j, _):
            for u in range(8):
                gseg_v[pl.ds((j * 8 + u) * LANES, LANES)] = zeros
            return _
        lax.fori_loop(0, SEG_PW // LANES // 8, zero_body, None)
        pltpu.sync_copy(gseg_v, den_sh.at[pl.ds(seg_lo, SEG_PW)])
        plsc.subcore_barrier()
        pltpu.sync_copy(gmax_sh, glob_v)

    with jax.named_scope("sc_phase_b"):
        def phase_b(i, _):
            for u in range(UNROLL):
                b = (i * UNROLL + u) * LANES
                s16 = src_v[pl.ds(b, LANES)]
                a16 = a_v[pl.ds(b, LANES)]
                mx = plsc.load_gather(glob_v, [s16])
                ex_v[pl.ds(b, LANES)] = jnp.exp(a16 - mx)
            return _
        lax.fori_loop(0, GROUPS // UNROLL, phase_b, None)
    with jax.named_scope("sc_scatter_add"):
        pltpu.sync_copy(ex_v, den_sh.at[src_v], add=True)
        plsc.subcore_barrier()
        pltpu.sync_copy(den_sh, glob_v)

    with jax.named_scope("sc_phase_c"):
        def phase_c(i, acc):
            for u in range(UNROLL):
                b = (i * UNROLL + u) * LANES
                s16 = src_v[pl.ds(b, LANES)]
                e16 = ex_v[pl.ds(b, LANES)]
                d16 = plsc.load_gather(glob_v, [s16])
                acc = acc + e16 / d16
            return acc
        acc = lax.fori_loop(0, GROUPS // UNROLL, phase_c,
                            jnp.zeros((LANES,), jnp.float32))
        accb[...] = acc
        pltpu.sync_copy(accb, out_hbm.at[wid])


_sc_softmax_partials = functools.partial(
    pl.kernel,
    mesh=plsc.VectorSubcoreMesh(core_axis_name="c", subcore_axis_name="s",
                                num_cores=1),
    compiler_params=pltpu.CompilerParams(needs_layout_passes=False),
    out_type=jax.ShapeDtypeStruct((N_WORKERS, LANES), jnp.float32),
    scratch_types=[
        pltpu.VMEM((EPW,), jnp.float32),
        pltpu.VMEM((EPW,), jnp.int32),
        pltpu.VMEM((EPW,), jnp.float32),
        pltpu.VMEM((NPAD,), jnp.float32),
        pltpu.VMEM((NPAD,), jnp.float32),
        pltpu.VMEM((N_WORKERS, SEG_PW), jnp.float32),
        pltpu.VMEM((SEG_PW,), jnp.float32),
        pltpu.VMEM((LANES,), jnp.int32),
        pltpu.VMEM((LANES,), jnp.int32),
        pltpu.VMEM((LANES,), jnp.int32),
        pltpu.VMEM((LANES,), jnp.int32),
        pltpu.VMEM((LANES,), jnp.int32),
        pltpu.VMEM((LANES,), jnp.float32),
        pltpu.VMEM((LANES,), jnp.float32),
        pltpu.VMEM((LANES,), jnp.float32),
        pltpu.VMEM((LANES,), jnp.float32),
        pltpu.VMEM((LANES,), jnp.float32),
        pltpu.VMEM((LANES,), jnp.float32),
        pltpu.VMEM_SHARED((N_WORKERS, NPAD), jnp.float32),
        pltpu.VMEM_SHARED((NPAD,), jnp.float32),
        pltpu.VMEM_SHARED((NPAD,), jnp.float32),
    ],
)(_sc_body)


def _sc_quick_body(a_hbm, src_hbm, out_hbm, bad_hbm,
                   a_v, src_v, ex_v, glob_v, gseg_v, vbuf, accb, badb, gmx_v,
                   gmax16_sh, den_sh):
    wid = lax.axis_index("s")
    base_e = wid * EPW
    pltpu.sync_copy(a_hbm.at[pl.ds(base_e, EPW)], a_v)
    pltpu.sync_copy(src_hbm.at[pl.ds(base_e, EPW)], src_v)

    neg = jnp.full((LANES,), NEG_INF, jnp.float32)
    zeros = jnp.zeros((LANES,), jnp.float32)

    def lmax_body(i, m):
        for u in range(UNROLL):
            m = jnp.maximum(m, a_v[pl.ds((i * UNROLL + u) * LANES, LANES)])
        return m
    m16 = lax.fori_loop(0, GROUPS // UNROLL, lmax_body, neg)
    vbuf[...] = m16
    pltpu.sync_copy(vbuf, gmax16_sh.at[wid])

    def zero_body(j, _):
        for u in range(8):
            gseg_v[pl.ds((j * 8 + u) * LANES, LANES)] = zeros
        return _
    lax.fori_loop(0, SEG_PW // LANES // 8, zero_body, None)
    pltpu.sync_copy(gseg_v, den_sh.at[pl.ds(wid * SEG_PW, SEG_PW)])
    plsc.subcore_barrier()
    pltpu.sync_copy(gmax16_sh, gmx_v)
    cm = neg
    for r in range(N_WORKERS):
        cm = jnp.maximum(cm, gmx_v[r, :])
    c_glob = lax.reduce_max(cm, axes=(0,))

    def qb_body(i, _):
        for u in range(UNROLL):
            b = (i * UNROLL + u) * LANES
            ex_v[pl.ds(b, LANES)] = jnp.exp(a_v[pl.ds(b, LANES)] - c_glob)
        return _
    lax.fori_loop(0, GROUPS // UNROLL, qb_body, None)
    pltpu.sync_copy(ex_v, den_sh.at[src_v], add=True)
    plsc.subcore_barrier()
    pltpu.sync_copy(den_sh, glob_v)

    def qc_body(i, carry):
        acc, bad = carry
        for u in range(UNROLL):
            b = (i * UNROLL + u) * LANES
            s16 = src_v[pl.ds(b, LANES)]
            e16 = ex_v[pl.ds(b, LANES)]
            d16 = plsc.load_gather(glob_v, [s16])
            ok = d16 > 0.0
            acc = acc + jnp.where(ok, e16 / d16, 0.0)
            bad = bad | ~ok
        return acc, bad
    acc, bad = lax.fori_loop(0, GROUPS // UNROLL, qc_body,
                             (jnp.zeros((LANES,), jnp.float32),
                              jnp.zeros((LANES,), jnp.bool_)))
    accb[...] = acc
    badb[...] = jnp.where(bad, 1.0, 0.0)
    pltpu.sync_copy(accb, out_hbm.at[wid])
    pltpu.sync_copy(badb, bad_hbm.at[wid])


_sc_quick = functools.partial(
    pl.kernel,
    mesh=plsc.VectorSubcoreMesh(core_axis_name="c", subcore_axis_name="s",
                                num_cores=1),
    compiler_params=pltpu.CompilerParams(needs_layout_passes=False),
    out_type=[
        jax.ShapeDtypeStruct((N_WORKERS, LANES), jnp.float32),
        jax.ShapeDtypeStruct((N_WORKERS, LANES), jnp.float32),
    ],
    scratch_types=[
        pltpu.VMEM((EPW,), jnp.float32),
        pltpu.VMEM((EPW,), jnp.int32),
        pltpu.VMEM((EPW,), jnp.float32),
        pltpu.VMEM((NPAD,), jnp.float32),
        pltpu.VMEM((SEG_PW,), jnp.float32),
        pltpu.VMEM((LANES,), jnp.float32),
        pltpu.VMEM((LANES,), jnp.float32),
        pltpu.VMEM((LANES,), jnp.float32),
        pltpu.VMEM((N_WORKERS, LANES), jnp.float32),
        pltpu.VMEM_SHARED((N_WORKERS, LANES), jnp.float32),
        pltpu.VMEM_SHARED((NPAD,), jnp.float32),
    ],
)(_sc_quick_body)


def _h_body(p_ref, x_ref, o_ref):
    o_ref[...] = x_ref[...] + 0.0 * jnp.sum(p_ref[...])


def kernel(node_features, edge_features, edge_index, W_attn):
    a, src = _edge_prep(edge_features, edge_index.astype(jnp.int32), W_attn)
    q_partials, badm = _sc_quick(a, src)
    partials = lax.cond(jnp.max(badm) > 0.0,
                        lambda: _sc_softmax_partials(a, src),
                        lambda: q_partials)

    rows, cols = node_features.shape
    blk = 2000
    h = pl.pallas_call(
        _h_body,
        grid=(rows // blk,),
        in_specs=[
            pl.BlockSpec((N_WORKERS, LANES), lambda i: (0, 0)),
            pl.BlockSpec((blk, cols), lambda i: (i, 0)),
        ],
        out_specs=pl.BlockSpec((blk, cols), lambda i: (i, 0)),
        out_shape=jax.ShapeDtypeStruct(node_features.shape,
                                       node_features.dtype),
    )(partials, node_features)
    return h
